# baseline (speedup 1.0000x reference)
import functools

import jax
import jax.numpy as jnp
from jax import lax
from jax.experimental import pallas as pl
from jax.experimental.pallas import tpu as pltpu
from jax.experimental.pallas import tpu_sc as plsc

N_NODES = 10000
N_EDGES = 320000
D = 128

NC = 2
NS = 16
NW = NC * NS
CHUNK = 128
NCHUNK = 80
EPW = NCHUNK * CHUNK
E_PAD = NW * EPW
NPAD = 10240
STRIPE = NPAD // NS
ZCH = 128

_mesh = plsc.VectorSubcoreMesh(core_axis_name="c", subcore_axis_name="s")


@functools.partial(
    pl.kernel,
    mesh=_mesh,
    out_type=jax.ShapeDtypeStruct((NC, NPAD, D), jnp.float32),
    scratch_types=[
        pltpu.VMEM((NCHUNK, CHUNK), jnp.int32),
        pltpu.VMEM((ZCH, D), jnp.float32),
        pltpu.VMEM_SHARED((NPAD, D), jnp.float32),
        pltpu.SemaphoreType.DMA,
    ],
)
def _sc_degree(dst_hbm, ones_hbm, zeros_hbm, out_hbm, idx_v, buf, acc_s, sem):
    c = lax.axis_index("c")
    s = lax.axis_index("s")
    wid = s * NC + c
    pltpu.sync_copy(dst_hbm.at[wid], idx_v)
    pltpu.sync_copy(zeros_hbm, buf)
    for r in range(STRIPE // ZCH):
        pltpu.sync_copy(buf, acc_s.at[pl.ds(s * STRIPE + r * ZCH, ZCH)])
    pltpu.sync_copy(ones_hbm, buf)
    plsc.subcore_barrier()

    DEPTH = 4
    for p in range(DEPTH):
        pltpu.async_copy(buf, acc_s.at[idx_v.at[p]], sem, add=True)

    def body(j, carry):
        pltpu.make_async_copy(buf, acc_s.at[idx_v.at[j]], sem).wait()
        pltpu.async_copy(buf, acc_s.at[idx_v.at[j + DEPTH]], sem, add=True)
        return carry

    lax.fori_loop(0, NCHUNK - DEPTH, body, 0)
    for p in range(DEPTH):
        pltpu.make_async_copy(buf, acc_s.at[idx_v.at[NCHUNK - DEPTH + p]], sem).wait()
    plsc.subcore_barrier()
    pltpu.sync_copy(
        acc_s.at[pl.ds(s * STRIPE, STRIPE)],
        out_hbm.at[c, pl.ds(s * STRIPE, STRIPE)],
    )


@functools.partial(
    pl.kernel,
    mesh=_mesh,
    out_type=jax.ShapeDtypeStruct((NC, NPAD, D), jnp.float32),
    scratch_types=[
        pltpu.VMEM((NCHUNK, CHUNK), jnp.int32),
        pltpu.VMEM((2, CHUNK), jnp.int32),
        pltpu.VMEM((CHUNK, D), jnp.float32),
        pltpu.VMEM((CHUNK, D), jnp.float32),
        pltpu.VMEM_SHARED((NPAD, D), jnp.float32),
        pltpu.SemaphoreType.DMA,
        pltpu.SemaphoreType.DMA,
    ],
)
def _sc_edge_accum(y_hbm, src_hbm, dst_hbm, zeros_hbm, out_hbm,
                   src_v, dst_v, buf0, buf1, acc_s, semg, semd):
    c = lax.axis_index("c")
    s = lax.axis_index("s")
    wid = s * NC + c
    pltpu.sync_copy(src_hbm.at[wid], src_v)
    pltpu.sync_copy(zeros_hbm, buf0)
    for r in range(STRIPE // CHUNK):
        pltpu.sync_copy(buf0, acc_s.at[pl.ds(s * STRIPE + r * CHUNK, CHUNK)])
    plsc.subcore_barrier()
    bufs = (buf0, buf1)

    pltpu.async_copy(y_hbm.at[src_v.at[0]], buf0, semg)
    pltpu.async_copy(dst_hbm.at[wid, 0], dst_v.at[0], semd)

    def body(g, carry):
        for b in range(2):
            j = g * 2 + b
            nxt = j + 1
            pltpu.make_async_copy(y_hbm.at[src_v.at[j]], bufs[b], semg).wait()

            @pl.when(nxt < NCHUNK)
            def _():
                pltpu.async_copy(y_hbm.at[src_v.at[nxt]], bufs[1 - b], semg)
                pltpu.async_copy(dst_hbm.at[wid, nxt], dst_v.at[1 - b], semd)

            pltpu.make_async_copy(dst_hbm.at[wid, j], dst_v.at[b], semd).wait()
            pltpu.sync_copy(bufs[b], acc_s.at[dst_v.at[b]], add=True)
        return carry

    lax.fori_loop(0, NCHUNK // 2, body, 0)
    plsc.subcore_barrier()
    pltpu.sync_copy(
        acc_s.at[pl.ds(s * STRIPE, STRIPE)],
        out_hbm.at[c, pl.ds(s * STRIPE, STRIPE)],
    )


def _tc_matmul_body(x_ref, w_ref, xw_ref):
    xw_ref[...] = jnp.dot(x_ref[...], w_ref[...], preferred_element_type=jnp.float32)


def _tc_prep_body(xw_ref, dp_ref, y_ref, dinv_ref):
    deg = 1.0 + dp_ref[0, :N_NODES, 0:1] + dp_ref[1, :N_NODES, 0:1]
    dinv = lax.rsqrt(deg)
    y_ref[...] = xw_ref[...] * dinv
    dinv_ref[...] = dinv


def _tc_final_body(acc_ref, y_ref, dinv_ref, b_ref, o_ref):
    o_ref[...] = (
        acc_ref[0, :N_NODES] + acc_ref[1, :N_NODES] + y_ref[...]
    ) * dinv_ref[...] + b_ref[...][None, :]


def kernel(x, edge_index, W, b):
    npad_e = E_PAD - N_EDGES
    pad_ids = jnp.arange(npad_e, dtype=edge_index.dtype)
    src = jnp.concatenate(
        [edge_index[0], pad_ids % N_NODES]
    ).reshape(NW, NCHUNK, CHUNK)
    dst = jnp.concatenate(
        [edge_index[1], N_NODES + pad_ids % (NPAD - N_NODES)]
    ).reshape(NW, NCHUNK, CHUNK)
    zerosZ = jnp.zeros((ZCH, D), jnp.float32)
    zerosC = jnp.zeros((CHUNK, D), jnp.float32)
    onesD = jnp.ones((ZCH, D), jnp.float32)

    deg_partial = _sc_degree(dst, onesD, zerosZ)

    xw = pl.pallas_call(
        _tc_matmul_body,
        out_shape=jax.ShapeDtypeStruct((N_NODES, D), jnp.float32),
    )(x, W)

    y, dinv = pl.pallas_call(
        _tc_prep_body,
        out_shape=[
            jax.ShapeDtypeStruct((N_NODES, D), jnp.float32),
            jax.ShapeDtypeStruct((N_NODES, 1), jnp.float32),
        ],
    )(xw, deg_partial)

    acc = _sc_edge_accum(y, src, dst, zerosC)

    out = pl.pallas_call(
        _tc_final_body,
        out_shape=jax.ShapeDtypeStruct((N_NODES, D), jnp.float32),
    )(acc, y, dinv, b)
    return out


# speedup vs baseline: 1.0153x; 1.0153x over previous
import functools

import jax
import jax.numpy as jnp
from jax import lax
from jax.experimental import pallas as pl
from jax.experimental.pallas import tpu as pltpu
from jax.experimental.pallas import tpu_sc as plsc

N_NODES = 10000
N_EDGES = 320000
D = 128

NC = 2
NS = 16
NW = NC * NS
CHUNK = 128
NCHUNK = 80
EPW = NCHUNK * CHUNK
E_PAD = NW * EPW
NPAD = 10240
STRIPE = NPAD // NS
ZCH = 128

_mesh = plsc.VectorSubcoreMesh(core_axis_name="c", subcore_axis_name="s")


@functools.partial(
    pl.kernel,
    mesh=_mesh,
    out_type=jax.ShapeDtypeStruct((NC, NPAD, D), jnp.float32),
    scratch_types=[
        pltpu.VMEM((NCHUNK, CHUNK), jnp.int32),
        pltpu.VMEM((ZCH, D), jnp.float32),
        pltpu.VMEM_SHARED((NPAD, D), jnp.float32),
        pltpu.SemaphoreType.DMA,
    ],
)
def _sc_degree(dst_hbm, ones_hbm, zeros_hbm, out_hbm, idx_v, buf, acc_s, sem):
    c = lax.axis_index("c")
    s = lax.axis_index("s")
    wid = s * NC + c
    pltpu.sync_copy(dst_hbm.at[wid], idx_v)
    pltpu.sync_copy(zeros_hbm, buf)
    for r in range(STRIPE // ZCH):
        pltpu.sync_copy(buf, acc_s.at[pl.ds(s * STRIPE + r * ZCH, ZCH)])
    pltpu.sync_copy(ones_hbm, buf)
    plsc.subcore_barrier()

    DEPTH = 4
    for p in range(DEPTH):
        pltpu.async_copy(buf, acc_s.at[idx_v.at[p]], sem, add=True)

    def body(j, carry):
        pltpu.make_async_copy(buf, acc_s.at[idx_v.at[j]], sem).wait()
        pltpu.async_copy(buf, acc_s.at[idx_v.at[j + DEPTH]], sem, add=True)
        return carry

    lax.fori_loop(0, NCHUNK - DEPTH, body, 0)
    for p in range(DEPTH):
        pltpu.make_async_copy(buf, acc_s.at[idx_v.at[NCHUNK - DEPTH + p]], sem).wait()
    plsc.subcore_barrier()
    pltpu.sync_copy(
        acc_s.at[pl.ds(s * STRIPE, STRIPE)],
        out_hbm.at[c, pl.ds(s * STRIPE, STRIPE)],
    )


@functools.partial(
    pl.kernel,
    mesh=_mesh,
    out_type=jax.ShapeDtypeStruct((NC, NPAD, D), jnp.float32),
    scratch_types=[
        pltpu.VMEM((NCHUNK, CHUNK), jnp.int32),
        pltpu.VMEM((2, CHUNK), jnp.int32),
        pltpu.VMEM((CHUNK, D), jnp.float32),
        pltpu.VMEM((CHUNK, D), jnp.float32),
        pltpu.VMEM_SHARED((NPAD, D), jnp.float32),
        pltpu.SemaphoreType.DMA,
        pltpu.SemaphoreType.DMA,
    ],
)
def _sc_edge_accum(y_hbm, src_hbm, dst_hbm, zeros_hbm, out_hbm,
                   src_v, dst_v, buf0, buf1, acc_s, semg, semd):
    c = lax.axis_index("c")
    s = lax.axis_index("s")
    wid = s * NC + c
    pltpu.sync_copy(src_hbm.at[wid], src_v)
    pltpu.sync_copy(zeros_hbm, buf0)
    for r in range(STRIPE // CHUNK):
        pltpu.sync_copy(buf0, acc_s.at[pl.ds(s * STRIPE + r * CHUNK, CHUNK)])
    plsc.subcore_barrier()
    bufs = (buf0, buf1)

    pltpu.async_copy(y_hbm.at[src_v.at[0]], buf0, semg)
    pltpu.async_copy(dst_hbm.at[wid, 0], dst_v.at[0], semd)

    def body(g, carry):
        for b in range(2):
            j = g * 2 + b
            nxt = j + 1
            pltpu.make_async_copy(y_hbm.at[src_v.at[j]], bufs[b], semg).wait()

            @pl.when(nxt < NCHUNK)
            def _():
                pltpu.async_copy(y_hbm.at[src_v.at[nxt]], bufs[1 - b], semg)
                pltpu.async_copy(dst_hbm.at[wid, nxt], dst_v.at[1 - b], semd)

            pltpu.make_async_copy(dst_hbm.at[wid, j], dst_v.at[b], semd).wait()
            pltpu.sync_copy(bufs[b], acc_s.at[dst_v.at[b]], add=True)
        return carry

    lax.fori_loop(0, NCHUNK // 2, body, 0)
    plsc.subcore_barrier()
    pltpu.sync_copy(
        acc_s.at[pl.ds(s * STRIPE, STRIPE)],
        out_hbm.at[c, pl.ds(s * STRIPE, STRIPE)],
    )


def _tc_prep_body(x_ref, w_ref, dp_ref, y_ref, dinv_ref):
    deg = 1.0 + dp_ref[0, :N_NODES, 0:1] + dp_ref[1, :N_NODES, 0:1]
    dinv = lax.rsqrt(deg)
    xw = jnp.dot(x_ref[...], w_ref[...], preferred_element_type=jnp.float32)
    y_ref[...] = xw * dinv
    dinv_ref[...] = dinv


def _tc_final_body(acc_ref, y_ref, dinv_ref, b_ref, o_ref):
    o_ref[...] = (
        acc_ref[0, :N_NODES] + acc_ref[1, :N_NODES] + y_ref[...]
    ) * dinv_ref[...] + b_ref[...][None, :]


def kernel(x, edge_index, W, b):
    npad_e = E_PAD - N_EDGES
    pad_ids = jnp.arange(npad_e, dtype=edge_index.dtype)
    src = jnp.concatenate(
        [edge_index[0], pad_ids % N_NODES]
    ).reshape(NW, NCHUNK, CHUNK)
    dst = jnp.concatenate(
        [edge_index[1], N_NODES + pad_ids % (NPAD - N_NODES)]
    ).reshape(NW, NCHUNK, CHUNK)
    zerosZ = jnp.zeros((ZCH, D), jnp.float32)
    zerosC = jnp.zeros((CHUNK, D), jnp.float32)
    onesD = jnp.ones((ZCH, D), jnp.float32)

    deg_partial = _sc_degree(dst, onesD, zerosZ)

    y, dinv = pl.pallas_call(
        _tc_prep_body,
        out_shape=[
            jax.ShapeDtypeStruct((N_NODES, D), jnp.float32),
            jax.ShapeDtypeStruct((N_NODES, 1), jnp.float32),
        ],
    )(x, W, deg_partial)

    acc = _sc_edge_accum(y, src, dst, zerosC)

    out = pl.pallas_call(
        _tc_final_body,
        out_shape=jax.ShapeDtypeStruct((N_NODES, D), jnp.float32),
    )(acc, y, dinv, b)
    return out
